# baseline (speedup 1.0000x reference)
import functools

import jax
import jax.numpy as jnp
from jax import lax
from jax.experimental import pallas as pl
from jax.experimental.pallas import tpu as pltpu
from jax.experimental.pallas import tpu_sc as plsc

NC, NS, L = 2, 16, 16
NW = NC * NS

SEQ = 200
BATCH = 16384
EMB = 32

CHUNK = 16
ROWS = CHUNK * SEQ
GATHER_W = 128
IDX_TILES = ROWS // GATHER_W
PER_W = BATCH // NW
N_CHUNKS = PER_W // CHUNK


def _sc_body(emb_h, idx_h, wb_h, bias_h, out_h,
             rowids_v, idx_v, idxf_v, rows_v, wb_v, bias_v, out_v, sem):
    w = lax.axis_index("s") * NC + lax.axis_index("c")

    pltpu.sync_copy(wb_h, wb_v)
    pltpu.sync_copy(bias_h, bias_v)

    lane = lax.iota(jnp.int32, L)
    zero = jnp.zeros((L,), jnp.float32)

    @pl.loop(0, N_CHUNKS)
    def chunk_loop(k):
        e = w * N_CHUNKS + k

        def gbody(g, _):
            rowids_v[pl.ds(g * L, L)] = (lane + g * L) * (BATCH // CHUNK) + e
            return 0

        lax.fori_loop(0, (SEQ + L - 1) // L, gbody, 0, unroll=13)
        cp0 = pltpu.async_copy(
            idx_h.at[rowids_v.at[pl.ds(0, GATHER_W)]],
            idx_v.at[pl.ds(0, GATHER_W), :], sem)
        cp1 = pltpu.async_copy(
            idx_h.at[rowids_v.at[pl.ds(GATHER_W, SEQ - GATHER_W)]],
            idx_v.at[pl.ds(GATHER_W, SEQ - GATHER_W), :], sem)
        cp0.wait()
        cp1.wait()

        def fbody(s, _):
            idxf_v[pl.ds(s * CHUNK, CHUNK)] = idx_v[s, 0:CHUNK]
            return 0

        lax.fori_loop(0, SEQ, fbody, 0, unroll=8)

        copies = []
        for j in range(IDX_TILES):
            copies.append(pltpu.async_copy(
                emb_h.at[idxf_v.at[pl.ds(j * GATHER_W, GATHER_W)]],
                rows_v.at[pl.ds(j * GATHER_W, GATHER_W), :],
                sem))
        for cp in copies:
            cp.wait()

        w0 = wb_v[0:16]
        w1 = wb_v[16:32]
        yacc = bias_v[...]
        for c in range(CHUNK):

            def rbody(r, acc, c=c):
                a0, a1 = acc
                a0 = a0 + rows_v[r * CHUNK + c, 0:16]
                a1 = a1 + rows_v[r * CHUNK + c, 16:32]
                return a0, a1

            a0, a1 = lax.fori_loop(0, SEQ, rbody, (zero, zero), unroll=8)
            e0 = jnp.exp(a0 * (2.0 / SEQ))
            e1 = jnp.exp(a1 * (2.0 / SEQ))
            t0 = 1.0 - 2.0 / (e0 + 1.0)
            t1 = 1.0 - 2.0 / (e1 + 1.0)
            total = jnp.sum(t0 * w0 + t1 * w1)
            yacc = yacc + jnp.where(lane == c, total, 0.0)
        out_v[pl.ds(k * CHUNK, CHUNK)] = yacc

    pltpu.sync_copy(out_v, out_h.at[pl.ds(w * PER_W, PER_W)])


@functools.partial(
    pl.kernel,
    out_type=jax.ShapeDtypeStruct((BATCH,), jnp.float32),
    mesh=plsc.VectorSubcoreMesh(core_axis_name="c", subcore_axis_name="s",
                                num_cores=NC, num_subcores=NS),
    compiler_params=pltpu.CompilerParams(needs_layout_passes=False,
                                         use_tc_tiling_on_sc=False),
    scratch_types=[
        pltpu.VMEM((((SEQ + L - 1) // L) * L,), jnp.int32),
        pltpu.VMEM((SEQ, CHUNK), jnp.int32),
        pltpu.VMEM((ROWS,), jnp.int32),
        pltpu.VMEM((ROWS, EMB), jnp.float32),
        pltpu.VMEM((EMB,), jnp.float32),
        pltpu.VMEM((L,), jnp.float32),
        pltpu.VMEM((PER_W,), jnp.float32),
        pltpu.SemaphoreType.DMA,
    ],
)
def _sc_kernel(emb_h, idx_h, wb_h, bias_h, out_h,
               rowids_v, idx_v, idxf_v, rows_v, wb_v, bias_v, out_v, sem):
    _sc_body(emb_h, idx_h, wb_h, bias_h, out_h,
             rowids_v, idx_v, idxf_v, rows_v, wb_v, bias_v, out_v, sem)


def kernel(input, emb, W, b):
    wb = W.reshape(EMB)
    bias = jnp.broadcast_to(b.reshape(1), (L,))
    idx16 = input.reshape(SEQ * BATCH // CHUNK, CHUNK)
    out = _sc_kernel(emb, idx16, wb, bias)
    return out.reshape(BATCH, 1)


# speedup vs baseline: 1.3242x; 1.3242x over previous
import functools

import jax
import jax.numpy as jnp
from jax import lax
from jax.experimental import pallas as pl
from jax.experimental.pallas import tpu as pltpu
from jax.experimental.pallas import tpu_sc as plsc
from jax.experimental.layout import Format, Layout, with_layout_constraint

NC, NS, L = 2, 16, 16
NW = NC * NS

SEQ = 200
BATCH = 16384
EMB = 32
VOCAB = 1000000

CHUNK = 16
ROWS = CHUNK * SEQ
GATHER_W = 128
IDX_TILES = ROWS // GATHER_W
PER_W = BATCH // NW
N_CHUNKS = PER_W // CHUNK


def _sc_body(emb_h, idx_h, wb_h, bias_h, out_h,
             rowids_v, idx_v, idxf_v, rows_v, wb_v, bias_v, out_v, sem):
    w = lax.axis_index("s") * NC + lax.axis_index("c")

    pltpu.sync_copy(wb_h, wb_v)
    pltpu.sync_copy(bias_h, bias_v)

    lane = lax.iota(jnp.int32, L)
    zero = jnp.zeros((L,), jnp.float32)

    @pl.loop(0, N_CHUNKS)
    def chunk_loop(k):
        e = w * N_CHUNKS + k

        def gbody(g, _):
            rowids_v[pl.ds(g * L, L)] = (lane + g * L) * (BATCH // CHUNK) + e
            return 0

        lax.fori_loop(0, (SEQ + L - 1) // L, gbody, 0, unroll=13)
        cp0 = pltpu.async_copy(
            idx_h.at[rowids_v.at[pl.ds(0, GATHER_W)]],
            idx_v.at[pl.ds(0, GATHER_W), :], sem)
        cp1 = pltpu.async_copy(
            idx_h.at[rowids_v.at[pl.ds(GATHER_W, SEQ - GATHER_W)]],
            idx_v.at[pl.ds(GATHER_W, SEQ - GATHER_W), :], sem)
        cp0.wait()
        cp1.wait()

        def fbody(s, _):
            idxf_v[pl.ds(s * CHUNK, CHUNK)] = idx_v[s, 0:CHUNK]
            return 0

        lax.fori_loop(0, SEQ, fbody, 0, unroll=8)

        copies = []
        for j in range(IDX_TILES):
            copies.append(pltpu.async_copy(
                emb_h.at[idxf_v.at[pl.ds(j * GATHER_W, GATHER_W)]],
                rows_v.at[pl.ds(j * GATHER_W, GATHER_W), :],
                sem))
        for cp in copies:
            cp.wait()

        w0 = wb_v[0:16]
        w1 = wb_v[16:32]
        yacc = bias_v[...]
        for c in range(CHUNK):

            def rbody(r, acc, c=c):
                a0, a1 = acc
                a0 = a0 + rows_v[r * CHUNK + c, 0:16]
                a1 = a1 + rows_v[r * CHUNK + c, 16:32]
                return a0, a1

            a0, a1 = lax.fori_loop(0, SEQ, rbody, (zero, zero), unroll=8)
            e0 = jnp.exp(a0 * (2.0 / SEQ))
            e1 = jnp.exp(a1 * (2.0 / SEQ))
            t0 = 1.0 - 2.0 / (e0 + 1.0)
            t1 = 1.0 - 2.0 / (e1 + 1.0)
            total = jnp.sum(t0 * w0 + t1 * w1)
            yacc = yacc + jnp.where(lane == c, total, 0.0)
        out_v[pl.ds(k * CHUNK, CHUNK)] = yacc

    pltpu.sync_copy(out_v, out_h.at[pl.ds(w * PER_W, PER_W)])


@functools.partial(
    pl.kernel,
    out_type=jax.ShapeDtypeStruct((BATCH,), jnp.float32),
    mesh=plsc.VectorSubcoreMesh(core_axis_name="c", subcore_axis_name="s",
                                num_cores=NC, num_subcores=NS),
    compiler_params=pltpu.CompilerParams(needs_layout_passes=False,
                                         use_tc_tiling_on_sc=False),
    scratch_types=[
        pltpu.VMEM((((SEQ + L - 1) // L) * L,), jnp.int32),
        pltpu.VMEM((SEQ, CHUNK), jnp.int32),
        pltpu.VMEM((ROWS,), jnp.int32),
        pltpu.VMEM((ROWS, EMB), jnp.float32),
        pltpu.VMEM((EMB,), jnp.float32),
        pltpu.VMEM((L,), jnp.float32),
        pltpu.VMEM((PER_W,), jnp.float32),
        pltpu.SemaphoreType.DMA,
    ],
)
def _sc_kernel(emb_h, idx_h, wb_h, bias_h, out_h,
               rowids_v, idx_v, idxf_v, rows_v, wb_v, bias_v, out_v, sem):
    _sc_body(emb_h, idx_h, wb_h, bias_h, out_h,
             rowids_v, idx_v, idxf_v, rows_v, wb_v, bias_v, out_v, sem)


def kernel(input, emb, W, b):
    wb = W.reshape(EMB)
    bias = jnp.broadcast_to(b.reshape(1), (L,))
    idx16 = input.reshape(SEQ * BATCH // CHUNK, CHUNK)
    emb_rm = with_layout_constraint(
        emb, Layout((0, 1), tiling=((8,), (1024,))))
    out = _sc_kernel(emb_rm, idx16, wb, bias)
    return out.reshape(BATCH, 1)


# speedup vs baseline: 1.5659x; 1.1825x over previous
import functools

import jax
import jax.numpy as jnp
from jax import lax
from jax.experimental import pallas as pl
from jax.experimental.pallas import tpu as pltpu
from jax.experimental.pallas import tpu_sc as plsc
from jax.experimental.layout import Format, Layout, with_layout_constraint

NC, NS, L = 2, 16, 16
NW = NC * NS

SEQ = 200
BATCH = 16384
EMB = 32
VOCAB = 1000000

TBV = 2048
NBLK = -(-VOCAB // (4 * TBV))
VOCAB_PAD = NBLK * 4 * TBV

CHUNK = 16
ROWS = CHUNK * SEQ
GATHER_W = 128
IDX_TILES = ROWS // GATHER_W
PER_W = BATCH // NW
N_CHUNKS = PER_W // CHUNK


def _sc_body(emb_h, idx_h, wb_h, bias_h, out_h,
             rowids_v, idx_v, idxf_v, rows_v, wb_v, bias_v, out_v, sem):
    w = lax.axis_index("s") * NC + lax.axis_index("c")

    pltpu.sync_copy(wb_h, wb_v)
    pltpu.sync_copy(bias_h, bias_v)

    lane = lax.iota(jnp.int32, L)
    zero = jnp.zeros((L,), jnp.float32)

    @pl.loop(0, N_CHUNKS)
    def chunk_loop(k):
        e = w * N_CHUNKS + k

        def gbody(g, _):
            rowids_v[pl.ds(g * L, L)] = (lane + g * L) * (BATCH // CHUNK) + e
            return 0

        lax.fori_loop(0, (SEQ + L - 1) // L, gbody, 0, unroll=13)
        cp0 = pltpu.async_copy(
            idx_h.at[rowids_v.at[pl.ds(0, GATHER_W)]],
            idx_v.at[pl.ds(0, GATHER_W), :], sem)
        cp1 = pltpu.async_copy(
            idx_h.at[rowids_v.at[pl.ds(GATHER_W, SEQ - GATHER_W)]],
            idx_v.at[pl.ds(GATHER_W, SEQ - GATHER_W), :], sem)
        cp0.wait()
        cp1.wait()

        def fbody(s, _):
            r = idx_v[s, 0:CHUNK]
            blk = r // (4 * TBV)
            rem = r - blk * (4 * TBV)
            c = rem // TBV
            t = rem - c * TBV
            idxf_v[pl.ds(s * CHUNK, CHUNK)] = blk * (4 * TBV) + t * 4 + c
            return 0

        lax.fori_loop(0, SEQ, fbody, 0, unroll=8)

        copies = []
        for j in range(IDX_TILES):
            copies.append(pltpu.async_copy(
                emb_h.at[idxf_v.at[pl.ds(j * GATHER_W, GATHER_W)]],
                rows_v.at[pl.ds(j * GATHER_W, GATHER_W), :],
                sem))
        for cp in copies:
            cp.wait()

        w0 = wb_v[0:16]
        w1 = wb_v[16:32]
        yacc = bias_v[...]
        for c in range(CHUNK):

            def rbody(r, acc, c=c):
                a0, a1 = acc
                a0 = a0 + rows_v[r * CHUNK + c, 0:16]
                a1 = a1 + rows_v[r * CHUNK + c, 16:32]
                return a0, a1

            a0, a1 = lax.fori_loop(0, SEQ, rbody, (zero, zero), unroll=8)
            e0 = jnp.exp(a0 * (2.0 / SEQ))
            e1 = jnp.exp(a1 * (2.0 / SEQ))
            t0 = 1.0 - 2.0 / (e0 + 1.0)
            t1 = 1.0 - 2.0 / (e1 + 1.0)
            total = jnp.sum(t0 * w0 + t1 * w1)
            yacc = yacc + jnp.where(lane == c, total, 0.0)
        out_v[pl.ds(k * CHUNK, CHUNK)] = yacc

    pltpu.sync_copy(out_v, out_h.at[pl.ds(w * PER_W, PER_W)])


@functools.partial(
    pl.kernel,
    out_type=jax.ShapeDtypeStruct((BATCH,), jnp.float32),
    mesh=plsc.VectorSubcoreMesh(core_axis_name="c", subcore_axis_name="s",
                                num_cores=NC, num_subcores=NS),
    compiler_params=pltpu.CompilerParams(needs_layout_passes=False,
                                         use_tc_tiling_on_sc=False),
    scratch_types=[
        pltpu.VMEM((((SEQ + L - 1) // L) * L,), jnp.int32),
        pltpu.VMEM((SEQ, CHUNK), jnp.int32),
        pltpu.VMEM((ROWS,), jnp.int32),
        pltpu.VMEM((ROWS, EMB), jnp.float32),
        pltpu.VMEM((EMB,), jnp.float32),
        pltpu.VMEM((L,), jnp.float32),
        pltpu.VMEM((PER_W,), jnp.float32),
        pltpu.SemaphoreType.DMA,
    ],
)
def _sc_kernel(emb_h, idx_h, wb_h, bias_h, out_h,
               rowids_v, idx_v, idxf_v, rows_v, wb_v, bias_v, out_v, sem):
    _sc_body(emb_h, idx_h, wb_h, bias_h, out_h,
             rowids_v, idx_v, idxf_v, rows_v, wb_v, bias_v, out_v, sem)


def _tc_transpose_body(in_ref, out_ref):
    x = in_ref[...]
    x128 = jnp.concatenate(
        [x[:, c * TBV:(c + 1) * TBV] for c in range(4)], axis=0)
    out_ref[...] = x128.T


_tc_transpose = pl.pallas_call(
    _tc_transpose_body,
    grid=(NBLK,),
    in_specs=[pl.BlockSpec((EMB, 4 * TBV), lambda i: (0, i))],
    out_specs=pl.BlockSpec((TBV, 128), lambda i: (i, 0)),
    out_shape=jax.ShapeDtypeStruct((NBLK * TBV, 128), jnp.float32),
)


def kernel(input, emb, W, b):
    wb = W.reshape(EMB)
    bias = jnp.broadcast_to(b.reshape(1), (L,))
    idx16 = input.reshape(SEQ * BATCH // CHUNK, CHUNK)
    emb_rm = _tc_transpose(emb.T).reshape(VOCAB_PAD, EMB)
    out = _sc_kernel(emb_rm, idx16, wb, bias)
    return out.reshape(BATCH, 1)


# speedup vs baseline: 1.6738x; 1.0689x over previous
import functools

import jax
import jax.numpy as jnp
from jax import lax
from jax.experimental import pallas as pl
from jax.experimental.pallas import tpu as pltpu
from jax.experimental.pallas import tpu_sc as plsc
from jax.experimental.layout import Format, Layout, with_layout_constraint

NC, NS, L = 2, 16, 16
NW = NC * NS

SEQ = 200
BATCH = 16384
EMB = 32
VOCAB = 1000000

TBV = 4096
NBLK = -(-VOCAB // (4 * TBV))
VOCAB_PAD = NBLK * 4 * TBV

CHUNK = 16
ROWS = CHUNK * SEQ
GATHER_W = 128
IDX_TILES = ROWS // GATHER_W
PER_W = BATCH // NW
N_CHUNKS = PER_W // CHUNK


def _sc_body(emb_h, idx_h, wb_h, bias_h, out_h,
             rowids_v, idx_v, idxf_v, rows_v, wb_v, bias_v, out_v, sem):
    w = lax.axis_index("s") * NC + lax.axis_index("c")

    pltpu.sync_copy(wb_h, wb_v)
    pltpu.sync_copy(bias_h, bias_v)

    lane = lax.iota(jnp.int32, L)
    zero = jnp.zeros((L,), jnp.float32)

    @pl.loop(0, N_CHUNKS)
    def chunk_loop(k):
        e = w * N_CHUNKS + k

        def gbody(g, _):
            rowids_v[pl.ds(g * L, L)] = (lane + g * L) * (BATCH // CHUNK) + e
            return 0

        lax.fori_loop(0, (SEQ + L - 1) // L, gbody, 0, unroll=13)
        cp0 = pltpu.async_copy(
            idx_h.at[rowids_v.at[pl.ds(0, GATHER_W)]],
            idx_v.at[pl.ds(0, GATHER_W), :], sem)
        cp1 = pltpu.async_copy(
            idx_h.at[rowids_v.at[pl.ds(GATHER_W, SEQ - GATHER_W)]],
            idx_v.at[pl.ds(GATHER_W, SEQ - GATHER_W), :], sem)
        cp0.wait()
        cp1.wait()

        def fbody(s, _):
            r = idx_v[s, 0:CHUNK]
            blk = r // (4 * TBV)
            rem = r - blk * (4 * TBV)
            c = rem // TBV
            t = rem - c * TBV
            idxf_v[pl.ds(s * CHUNK, CHUNK)] = blk * (4 * TBV) + t * 4 + c
            return 0

        lax.fori_loop(0, SEQ, fbody, 0, unroll=8)

        copies = []
        for j in range(IDX_TILES):
            copies.append(pltpu.async_copy(
                emb_h.at[idxf_v.at[pl.ds(j * GATHER_W, GATHER_W)]],
                rows_v.at[pl.ds(j * GATHER_W, GATHER_W), :],
                sem))
        for cp in copies:
            cp.wait()

        w0 = wb_v[0:16]
        w1 = wb_v[16:32]
        yacc = bias_v[...]
        for c in range(CHUNK):

            def rbody(r, acc, c=c):
                a0, a1 = acc
                a0 = a0 + rows_v[r * CHUNK + c, 0:16]
                a1 = a1 + rows_v[r * CHUNK + c, 16:32]
                return a0, a1

            a0, a1 = lax.fori_loop(0, SEQ, rbody, (zero, zero), unroll=8)
            e0 = jnp.exp(a0 * (2.0 / SEQ))
            e1 = jnp.exp(a1 * (2.0 / SEQ))
            t0 = 1.0 - 2.0 / (e0 + 1.0)
            t1 = 1.0 - 2.0 / (e1 + 1.0)
            total = jnp.sum(t0 * w0 + t1 * w1)
            yacc = yacc + jnp.where(lane == c, total, 0.0)
        out_v[pl.ds(k * CHUNK, CHUNK)] = yacc

    pltpu.sync_copy(out_v, out_h.at[pl.ds(w * PER_W, PER_W)])


@functools.partial(
    pl.kernel,
    out_type=jax.ShapeDtypeStruct((BATCH,), jnp.float32),
    mesh=plsc.VectorSubcoreMesh(core_axis_name="c", subcore_axis_name="s",
                                num_cores=NC, num_subcores=NS),
    compiler_params=pltpu.CompilerParams(needs_layout_passes=False,
                                         use_tc_tiling_on_sc=False),
    scratch_types=[
        pltpu.VMEM((((SEQ + L - 1) // L) * L,), jnp.int32),
        pltpu.VMEM((SEQ, CHUNK), jnp.int32),
        pltpu.VMEM((ROWS,), jnp.int32),
        pltpu.VMEM((ROWS, EMB), jnp.float32),
        pltpu.VMEM((EMB,), jnp.float32),
        pltpu.VMEM((L,), jnp.float32),
        pltpu.VMEM((PER_W,), jnp.float32),
        pltpu.SemaphoreType.DMA,
    ],
)
def _sc_kernel(emb_h, idx_h, wb_h, bias_h, out_h,
               rowids_v, idx_v, idxf_v, rows_v, wb_v, bias_v, out_v, sem):
    _sc_body(emb_h, idx_h, wb_h, bias_h, out_h,
             rowids_v, idx_v, idxf_v, rows_v, wb_v, bias_v, out_v, sem)


def _tc_transpose_body(in_ref, out_ref):
    x = in_ref[...]
    x128 = jnp.concatenate(
        [x[:, c * TBV:(c + 1) * TBV] for c in range(4)], axis=0)
    out_ref[...] = x128.T


_tc_transpose = pl.pallas_call(
    _tc_transpose_body,
    grid=(NBLK,),
    in_specs=[pl.BlockSpec((EMB, 4 * TBV), lambda i: (0, i))],
    out_specs=pl.BlockSpec((TBV, 128), lambda i: (i, 0)),
    out_shape=jax.ShapeDtypeStruct((NBLK * TBV, 128), jnp.float32),
)


def kernel(input, emb, W, b):
    wb = W.reshape(EMB)
    bias = jnp.broadcast_to(b.reshape(1), (L,))
    idx16 = input.reshape(SEQ * BATCH // CHUNK, CHUNK)
    emb_rm = _tc_transpose(emb.T).reshape(VOCAB_PAD, EMB)
    out = _sc_kernel(emb_rm, idx16, wb, bias)
    return out.reshape(BATCH, 1)


# speedup vs baseline: 2.0560x; 1.2283x over previous
import functools

import jax
import jax.numpy as jnp
from jax import lax
from jax.experimental import pallas as pl
from jax.experimental.pallas import tpu as pltpu
from jax.experimental.pallas import tpu_sc as plsc
from jax.experimental.layout import Format, Layout, with_layout_constraint

NC, NS, L = 2, 16, 16
NW = NC * NS

SEQ = 200
BATCH = 16384
EMB = 32
VOCAB = 1000000

TBV = 4096
NBLK = -(-VOCAB // (4 * TBV))
VOCAB_PAD = NBLK * 4 * TBV

CHUNK = 16
ROWS = CHUNK * SEQ
GATHER_W = 128
SEQ_A = 104
SEQ_B = SEQ - SEQ_A
HALF_A = SEQ_A * CHUNK
HALF_B = SEQ_B * CHUNK
PER_W = BATCH // NW
N_CHUNKS = PER_W // CHUNK


def _sc_body(emb_h, idx_h, wb_h, bias_h, out_h,
             rowids_v, idx_v, idxf_v, rows_v, acc_v, wb_v, bias_v, out_v, sem):
    w = lax.axis_index("s") * NC + lax.axis_index("c")

    pltpu.sync_copy(wb_h, wb_v)
    pltpu.sync_copy(bias_h, bias_v)

    lane = lax.iota(jnp.int32, L)
    zero = jnp.zeros((L,), jnp.float32)

    def prep(k):
        e = lax.min(w * N_CHUNKS + k, NW * N_CHUNKS - 1)

        def gbody(g, _):
            rowids_v[pl.ds(g * L, L)] = (lane + g * L) * (BATCH // CHUNK) + e
            return 0

        lax.fori_loop(0, (SEQ + L - 1) // L, gbody, 0, unroll=13)
        cp0 = pltpu.async_copy(
            idx_h.at[rowids_v.at[pl.ds(0, GATHER_W)]],
            idx_v.at[pl.ds(0, GATHER_W), :], sem)
        cp1 = pltpu.async_copy(
            idx_h.at[rowids_v.at[pl.ds(GATHER_W, SEQ - GATHER_W)]],
            idx_v.at[pl.ds(GATHER_W, SEQ - GATHER_W), :], sem)
        cp0.wait()
        cp1.wait()

        def fbody(s, _):
            r = idx_v[s, 0:CHUNK]
            blk = r // (4 * TBV)
            rem = r - blk * (4 * TBV)
            c = rem // TBV
            t = rem - c * TBV
            idxf_v[pl.ds(s * CHUNK, CHUNK)] = blk * (4 * TBV) + t * 4 + c
            return 0

        lax.fori_loop(0, SEQ, fbody, 0, unroll=8)

    def issue_a():
        return [pltpu.async_copy(
            emb_h.at[idxf_v.at[pl.ds(j * GATHER_W, GATHER_W)]],
            rows_v.at[0, pl.ds(j * GATHER_W, GATHER_W), :],
            sem) for j in range(HALF_A // GATHER_W)]

    def issue_b():
        return [pltpu.async_copy(
            emb_h.at[idxf_v.at[pl.ds(HALF_A + j * GATHER_W, GATHER_W)]],
            rows_v.at[1, pl.ds(j * GATHER_W, GATHER_W), :],
            sem) for j in range(HALF_B // GATHER_W)]

    def accumulate_a():
        for c in range(CHUNK):

            def rbody(r, acc, c=c):
                a0, a1 = acc
                a0 = a0 + rows_v[0, r * CHUNK + c, 0:16]
                a1 = a1 + rows_v[0, r * CHUNK + c, 16:32]
                return a0, a1

            a0, a1 = lax.fori_loop(0, SEQ_A, rbody, (zero, zero), unroll=8)
            acc_v[c, :] = a0
            acc_v[CHUNK + c, :] = a1

    def accumulate_b(k):
        w0 = wb_v[0:16]
        w1 = wb_v[16:32]
        yacc = bias_v[...]
        for c in range(CHUNK):

            def rbody(r, acc, c=c):
                a0, a1 = acc
                a0 = a0 + rows_v[1, r * CHUNK + c, 0:16]
                a1 = a1 + rows_v[1, r * CHUNK + c, 16:32]
                return a0, a1

            a0, a1 = lax.fori_loop(0, SEQ_B, rbody,
                                   (acc_v[c, :], acc_v[CHUNK + c, :]),
                                   unroll=8)
            e0 = jnp.exp(a0 * (2.0 / SEQ))
            e1 = jnp.exp(a1 * (2.0 / SEQ))
            t0 = 1.0 - 2.0 / (e0 + 1.0)
            t1 = 1.0 - 2.0 / (e1 + 1.0)
            total = jnp.sum(t0 * w0 + t1 * w1)
            yacc = yacc + jnp.where(lane == c, total, 0.0)
        out_v[pl.ds(k * CHUNK, CHUNK)] = yacc

    prep(0)
    for cp in issue_a():
        cp.wait()

    @pl.loop(0, N_CHUNKS)
    def chunk_loop(k):
        cps_b = issue_b()
        accumulate_a()
        for cp in cps_b:
            cp.wait()
        prep(k + 1)
        cps_a = issue_a()
        accumulate_b(k)
        for cp in cps_a:
            cp.wait()

    pltpu.sync_copy(out_v, out_h.at[pl.ds(w * PER_W, PER_W)])


@functools.partial(
    pl.kernel,
    out_type=jax.ShapeDtypeStruct((BATCH,), jnp.float32),
    mesh=plsc.VectorSubcoreMesh(core_axis_name="c", subcore_axis_name="s",
                                num_cores=NC, num_subcores=NS),
    compiler_params=pltpu.CompilerParams(needs_layout_passes=False,
                                         use_tc_tiling_on_sc=False),
    scratch_types=[
        pltpu.VMEM((((SEQ + L - 1) // L) * L,), jnp.int32),
        pltpu.VMEM((SEQ, CHUNK), jnp.int32),
        pltpu.VMEM((ROWS,), jnp.int32),
        pltpu.VMEM((2, HALF_A, EMB), jnp.float32),
        pltpu.VMEM((2 * CHUNK, L), jnp.float32),
        pltpu.VMEM((EMB,), jnp.float32),
        pltpu.VMEM((L,), jnp.float32),
        pltpu.VMEM((PER_W,), jnp.float32),
        pltpu.SemaphoreType.DMA,
    ],
)
def _sc_kernel(emb_h, idx_h, wb_h, bias_h, out_h,
               rowids_v, idx_v, idxf_v, rows_v, acc_v, wb_v, bias_v, out_v,
               sem):
    _sc_body(emb_h, idx_h, wb_h, bias_h, out_h,
             rowids_v, idx_v, idxf_v, rows_v, acc_v, wb_v, bias_v, out_v, sem)


def _tc_transpose_body(in_ref, out_ref):
    x = in_ref[...]
    x128 = jnp.concatenate(
        [x[:, c * TBV:(c + 1) * TBV] for c in range(4)], axis=0)
    out_ref[...] = x128.T


_tc_transpose = pl.pallas_call(
    _tc_transpose_body,
    grid=(NBLK,),
    in_specs=[pl.BlockSpec((EMB, 4 * TBV), lambda i: (0, i))],
    out_specs=pl.BlockSpec((TBV, 128), lambda i: (i, 0)),
    out_shape=jax.ShapeDtypeStruct((NBLK * TBV, 128), jnp.float32),
)


def kernel(input, emb, W, b):
    wb = W.reshape(EMB)
    bias = jnp.broadcast_to(b.reshape(1), (L,))
    idx16 = input.reshape(SEQ * BATCH // CHUNK, CHUNK)
    emb_rm = _tc_transpose(emb.T).reshape(VOCAB_PAD, EMB)
    out = _sc_kernel(emb_rm, idx16, wb, bias)
    return out.reshape(BATCH, 1)


# speedup vs baseline: 2.1167x; 1.0295x over previous
import functools

import jax
import jax.numpy as jnp
from jax import lax
from jax.experimental import pallas as pl
from jax.experimental.pallas import tpu as pltpu
from jax.experimental.pallas import tpu_sc as plsc
from jax.experimental.layout import Format, Layout, with_layout_constraint

NC, NS, L = 2, 16, 16
NW = NC * NS

SEQ = 200
BATCH = 16384
EMB = 32
VOCAB = 1000000

TBV = 8192
NBLK = -(-VOCAB // (4 * TBV))
VOCAB_PAD = NBLK * 4 * TBV

CHUNK = 16
ROWS = CHUNK * SEQ
GATHER_W = 128
SEQ_A = 104
SEQ_B = SEQ - SEQ_A
HALF_A = SEQ_A * CHUNK
HALF_B = SEQ_B * CHUNK
PER_W = BATCH // NW
N_CHUNKS = PER_W // CHUNK


def _sc_body(emb_h, idx_h, wb_h, bias_h, out_h,
             rowids_v, idx_v, idxf_v, rows_v, acc_v, wb_v, bias_v, out_v, sem):
    w = lax.axis_index("s") * NC + lax.axis_index("c")

    pltpu.sync_copy(wb_h, wb_v)
    pltpu.sync_copy(bias_h, bias_v)

    lane = lax.iota(jnp.int32, L)
    zero = jnp.zeros((L,), jnp.float32)

    def prep(k):
        e = lax.min(w * N_CHUNKS + k, NW * N_CHUNKS - 1)

        def gbody(g, _):
            rowids_v[pl.ds(g * L, L)] = (lane + g * L) * (BATCH // CHUNK) + e
            return 0

        lax.fori_loop(0, (SEQ + L - 1) // L, gbody, 0, unroll=13)
        cp0 = pltpu.async_copy(
            idx_h.at[rowids_v.at[pl.ds(0, GATHER_W)]],
            idx_v.at[pl.ds(0, GATHER_W), :], sem)
        cp1 = pltpu.async_copy(
            idx_h.at[rowids_v.at[pl.ds(GATHER_W, SEQ - GATHER_W)]],
            idx_v.at[pl.ds(GATHER_W, SEQ - GATHER_W), :], sem)
        cp0.wait()
        cp1.wait()

        def fbody(s, _):
            r = idx_v[s, 0:CHUNK]
            blk = r // (4 * TBV)
            rem = r - blk * (4 * TBV)
            c = rem // TBV
            t = rem - c * TBV
            idxf_v[pl.ds(s * CHUNK, CHUNK)] = blk * (4 * TBV) + t * 4 + c
            return 0

        lax.fori_loop(0, SEQ, fbody, 0, unroll=8)

    def issue_a():
        return [pltpu.async_copy(
            emb_h.at[idxf_v.at[pl.ds(j * GATHER_W, GATHER_W)]],
            rows_v.at[0, pl.ds(j * GATHER_W, GATHER_W), :],
            sem) for j in range(HALF_A // GATHER_W)]

    def issue_b():
        return [pltpu.async_copy(
            emb_h.at[idxf_v.at[pl.ds(HALF_A + j * GATHER_W, GATHER_W)]],
            rows_v.at[1, pl.ds(j * GATHER_W, GATHER_W), :],
            sem) for j in range(HALF_B // GATHER_W)]

    def accumulate_a():
        for c in range(CHUNK):

            def rbody(r, acc, c=c):
                a0, a1 = acc
                a0 = a0 + rows_v[0, r * CHUNK + c, 0:16]
                a1 = a1 + rows_v[0, r * CHUNK + c, 16:32]
                return a0, a1

            a0, a1 = lax.fori_loop(0, SEQ_A, rbody, (zero, zero), unroll=8)
            acc_v[c, :] = a0
            acc_v[CHUNK + c, :] = a1

    def accumulate_b(k):
        w0 = wb_v[0:16]
        w1 = wb_v[16:32]
        yacc = bias_v[...]
        for c in range(CHUNK):

            def rbody(r, acc, c=c):
                a0, a1 = acc
                a0 = a0 + rows_v[1, r * CHUNK + c, 0:16]
                a1 = a1 + rows_v[1, r * CHUNK + c, 16:32]
                return a0, a1

            a0, a1 = lax.fori_loop(0, SEQ_B, rbody,
                                   (acc_v[c, :], acc_v[CHUNK + c, :]),
                                   unroll=8)
            e0 = jnp.exp(a0 * (2.0 / SEQ))
            e1 = jnp.exp(a1 * (2.0 / SEQ))
            t0 = 1.0 - 2.0 / (e0 + 1.0)
            t1 = 1.0 - 2.0 / (e1 + 1.0)
            total = jnp.sum(t0 * w0 + t1 * w1)
            yacc = yacc + jnp.where(lane == c, total, 0.0)
        out_v[pl.ds(k * CHUNK, CHUNK)] = yacc

    prep(0)
    for cp in issue_a():
        cp.wait()

    @pl.loop(0, N_CHUNKS)
    def chunk_loop(k):
        cps_b = issue_b()
        accumulate_a()
        for cp in cps_b:
            cp.wait()
        prep(k + 1)
        cps_a = issue_a()
        accumulate_b(k)
        for cp in cps_a:
            cp.wait()

    pltpu.sync_copy(out_v, out_h.at[pl.ds(w * PER_W, PER_W)])


@functools.partial(
    pl.kernel,
    out_type=jax.ShapeDtypeStruct((BATCH,), jnp.float32),
    mesh=plsc.VectorSubcoreMesh(core_axis_name="c", subcore_axis_name="s",
                                num_cores=NC, num_subcores=NS),
    compiler_params=pltpu.CompilerParams(needs_layout_passes=False,
                                         use_tc_tiling_on_sc=False),
    scratch_types=[
        pltpu.VMEM((((SEQ + L - 1) // L) * L,), jnp.int32),
        pltpu.VMEM((SEQ, CHUNK), jnp.int32),
        pltpu.VMEM((ROWS,), jnp.int32),
        pltpu.VMEM((2, HALF_A, EMB), jnp.float32),
        pltpu.VMEM((2 * CHUNK, L), jnp.float32),
        pltpu.VMEM((EMB,), jnp.float32),
        pltpu.VMEM((L,), jnp.float32),
        pltpu.VMEM((PER_W,), jnp.float32),
        pltpu.SemaphoreType.DMA,
    ],
)
def _sc_kernel(emb_h, idx_h, wb_h, bias_h, out_h,
               rowids_v, idx_v, idxf_v, rows_v, acc_v, wb_v, bias_v, out_v,
               sem):
    _sc_body(emb_h, idx_h, wb_h, bias_h, out_h,
             rowids_v, idx_v, idxf_v, rows_v, acc_v, wb_v, bias_v, out_v, sem)


def _tc_transpose_body(in_ref, out_ref):
    x = in_ref[...]
    x128 = jnp.concatenate(
        [x[:, c * TBV:(c + 1) * TBV] for c in range(4)], axis=0)
    out_ref[...] = x128.T


_tc_transpose = pl.pallas_call(
    _tc_transpose_body,
    grid=(NBLK,),
    in_specs=[pl.BlockSpec((EMB, 4 * TBV), lambda i: (0, i))],
    out_specs=pl.BlockSpec((TBV, 128), lambda i: (i, 0)),
    out_shape=jax.ShapeDtypeStruct((NBLK * TBV, 128), jnp.float32),
)


def kernel(input, emb, W, b):
    wb = W.reshape(EMB)
    bias = jnp.broadcast_to(b.reshape(1), (L,))
    idx16 = input.reshape(SEQ * BATCH // CHUNK, CHUNK)
    emb_rm = _tc_transpose(emb.T).reshape(VOCAB_PAD, EMB)
    out = _sc_kernel(emb_rm, idx16, wb, bias)
    return out.reshape(BATCH, 1)


# speedup vs baseline: 2.1300x; 1.0063x over previous
import functools

import jax
import jax.numpy as jnp
from jax import lax
from jax.experimental import pallas as pl
from jax.experimental.pallas import tpu as pltpu
from jax.experimental.pallas import tpu_sc as plsc
from jax.experimental.layout import Format, Layout, with_layout_constraint

NC, NS, L = 2, 16, 16
NW = NC * NS

SEQ = 200
BATCH = 16384
EMB = 32
VOCAB = 1000000

TBV = 16384
NBLK = -(-VOCAB // (4 * TBV))
VOCAB_PAD = NBLK * 4 * TBV

CHUNK = 16
ROWS = CHUNK * SEQ
GATHER_W = 128
SEQ_A = 104
SEQ_B = SEQ - SEQ_A
HALF_A = SEQ_A * CHUNK
HALF_B = SEQ_B * CHUNK
PER_W = BATCH // NW
N_CHUNKS = PER_W // CHUNK


def _sc_body(emb_h, idx_h, wb_h, bias_h, out_h,
             rowids_v, idx_v, idxf_v, rows_v, acc_v, wb_v, bias_v, out_v, sem):
    w = lax.axis_index("s") * NC + lax.axis_index("c")

    pltpu.sync_copy(wb_h, wb_v)
    pltpu.sync_copy(bias_h, bias_v)

    lane = lax.iota(jnp.int32, L)
    zero = jnp.zeros((L,), jnp.float32)

    def prep(k):
        e = lax.min(w * N_CHUNKS + k, NW * N_CHUNKS - 1)

        def gbody(g, _):
            rowids_v[pl.ds(g * L, L)] = (lane + g * L) * (BATCH // CHUNK) + e
            return 0

        lax.fori_loop(0, (SEQ + L - 1) // L, gbody, 0, unroll=13)
        cp0 = pltpu.async_copy(
            idx_h.at[rowids_v.at[pl.ds(0, GATHER_W)]],
            idx_v.at[pl.ds(0, GATHER_W), :], sem)
        cp1 = pltpu.async_copy(
            idx_h.at[rowids_v.at[pl.ds(GATHER_W, SEQ - GATHER_W)]],
            idx_v.at[pl.ds(GATHER_W, SEQ - GATHER_W), :], sem)
        cp0.wait()
        cp1.wait()

        def fbody(s, _):
            r = idx_v[s, 0:CHUNK]
            blk = r // (4 * TBV)
            rem = r - blk * (4 * TBV)
            c = rem // TBV
            t = rem - c * TBV
            idxf_v[pl.ds(s * CHUNK, CHUNK)] = blk * (4 * TBV) + t * 4 + c
            return 0

        lax.fori_loop(0, SEQ, fbody, 0, unroll=8)

    def issue_a():
        return [pltpu.async_copy(
            emb_h.at[idxf_v.at[pl.ds(j * GATHER_W, GATHER_W)]],
            rows_v.at[0, pl.ds(j * GATHER_W, GATHER_W), :],
            sem) for j in range(HALF_A // GATHER_W)]

    def issue_b():
        return [pltpu.async_copy(
            emb_h.at[idxf_v.at[pl.ds(HALF_A + j * GATHER_W, GATHER_W)]],
            rows_v.at[1, pl.ds(j * GATHER_W, GATHER_W), :],
            sem) for j in range(HALF_B // GATHER_W)]

    def accumulate_a():
        for c in range(CHUNK):

            def rbody(r, acc, c=c):
                a0, a1 = acc
                a0 = a0 + rows_v[0, r * CHUNK + c, 0:16]
                a1 = a1 + rows_v[0, r * CHUNK + c, 16:32]
                return a0, a1

            a0, a1 = lax.fori_loop(0, SEQ_A, rbody, (zero, zero), unroll=8)
            acc_v[c, :] = a0
            acc_v[CHUNK + c, :] = a1

    def accumulate_b(k):
        w0 = wb_v[0:16]
        w1 = wb_v[16:32]
        yacc = bias_v[...]
        for c in range(CHUNK):

            def rbody(r, acc, c=c):
                a0, a1 = acc
                a0 = a0 + rows_v[1, r * CHUNK + c, 0:16]
                a1 = a1 + rows_v[1, r * CHUNK + c, 16:32]
                return a0, a1

            a0, a1 = lax.fori_loop(0, SEQ_B, rbody,
                                   (acc_v[c, :], acc_v[CHUNK + c, :]),
                                   unroll=8)
            e0 = jnp.exp(a0 * (2.0 / SEQ))
            e1 = jnp.exp(a1 * (2.0 / SEQ))
            t0 = 1.0 - 2.0 / (e0 + 1.0)
            t1 = 1.0 - 2.0 / (e1 + 1.0)
            total = jnp.sum(t0 * w0 + t1 * w1)
            yacc = yacc + jnp.where(lane == c, total, 0.0)
        out_v[pl.ds(k * CHUNK, CHUNK)] = yacc

    prep(0)
    for cp in issue_a():
        cp.wait()

    @pl.loop(0, N_CHUNKS)
    def chunk_loop(k):
        cps_b = issue_b()
        accumulate_a()
        for cp in cps_b:
            cp.wait()
        prep(k + 1)
        cps_a = issue_a()
        accumulate_b(k)
        for cp in cps_a:
            cp.wait()

    pltpu.sync_copy(out_v, out_h.at[pl.ds(w * PER_W, PER_W)])


@functools.partial(
    pl.kernel,
    out_type=jax.ShapeDtypeStruct((BATCH,), jnp.float32),
    mesh=plsc.VectorSubcoreMesh(core_axis_name="c", subcore_axis_name="s",
                                num_cores=NC, num_subcores=NS),
    compiler_params=pltpu.CompilerParams(needs_layout_passes=False,
                                         use_tc_tiling_on_sc=False),
    scratch_types=[
        pltpu.VMEM((((SEQ + L - 1) // L) * L,), jnp.int32),
        pltpu.VMEM((SEQ, CHUNK), jnp.int32),
        pltpu.VMEM((ROWS,), jnp.int32),
        pltpu.VMEM((2, HALF_A, EMB), jnp.float32),
        pltpu.VMEM((2 * CHUNK, L), jnp.float32),
        pltpu.VMEM((EMB,), jnp.float32),
        pltpu.VMEM((L,), jnp.float32),
        pltpu.VMEM((PER_W,), jnp.float32),
        pltpu.SemaphoreType.DMA,
    ],
)
def _sc_kernel(emb_h, idx_h, wb_h, bias_h, out_h,
               rowids_v, idx_v, idxf_v, rows_v, acc_v, wb_v, bias_v, out_v,
               sem):
    _sc_body(emb_h, idx_h, wb_h, bias_h, out_h,
             rowids_v, idx_v, idxf_v, rows_v, acc_v, wb_v, bias_v, out_v, sem)


def _tc_transpose_body(in_ref, out_ref):
    x = in_ref[...]
    x128 = jnp.concatenate(
        [x[:, c * TBV:(c + 1) * TBV] for c in range(4)], axis=0)
    out_ref[...] = x128.T


_tc_transpose = pl.pallas_call(
    _tc_transpose_body,
    grid=(NBLK,),
    in_specs=[pl.BlockSpec((EMB, 4 * TBV), lambda i: (0, i))],
    out_specs=pl.BlockSpec((TBV, 128), lambda i: (i, 0)),
    out_shape=jax.ShapeDtypeStruct((NBLK * TBV, 128), jnp.float32),
)


def kernel(input, emb, W, b):
    wb = W.reshape(EMB)
    bias = jnp.broadcast_to(b.reshape(1), (L,))
    idx16 = input.reshape(SEQ * BATCH // CHUNK, CHUNK)
    emb_rm = _tc_transpose(emb.T).reshape(VOCAB_PAD, EMB)
    out = _sc_kernel(emb_rm, idx16, wb, bias)
    return out.reshape(BATCH, 1)
